# Initial kernel scaffold; baseline (speedup 1.0000x reference)
import jax
import jax.numpy as jnp
from jax.experimental import pallas as pl


def kernel(atom_fea, crystal_atom_idx, W1, b1, W2, b2, Wp, bp):
    raise NotImplementedError("write your pallas kernel here")


# speedup vs baseline: 2.8037x; 2.8037x over previous
import functools

import jax
import jax.numpy as jnp
from jax import lax
from jax.experimental import pallas as pl
from jax.experimental.pallas import tpu as pltpu
from jax.experimental.pallas import tpu_sc as plsc


def _sc_gather(table, idx_flat):
    n_rows, d = table.shape
    r = idx_flat.shape[0]
    nc, ns = 2, 16
    nw = nc * ns
    chunk = 128
    per_w = r // nw
    j_per_w = per_w // chunk
    idx2 = idx_flat.reshape(r // chunk, chunk)

    mesh = plsc.VectorSubcoreMesh(core_axis_name="c", subcore_axis_name="s")

    @functools.partial(
        pl.kernel,
        out_type=jax.ShapeDtypeStruct((r, d), jnp.float32),
        mesh=mesh,
        scratch_types=[
            pltpu.VMEM((j_per_w, chunk), jnp.int32),
            pltpu.VMEM((chunk, d), jnp.float32),
            pltpu.SemaphoreType.DMA,
        ],
    )
    def gather_kernel(table_hbm, idx_hbm, out_hbm, idx_v, rows_v, sem):
        wid = lax.axis_index("s") * nc + lax.axis_index("c")
        pltpu.sync_copy(idx_hbm.at[pl.ds(wid * j_per_w, j_per_w)], idx_v)

        def body(j, carry):
            pltpu.async_copy(table_hbm.at[idx_v.at[j]], rows_v, sem).wait()
            pltpu.sync_copy(
                rows_v, out_hbm.at[pl.ds(wid * per_w + j * chunk, chunk)])
            return carry

        lax.fori_loop(0, j_per_w, body, 0)

    return gather_kernel(table, idx2)


def _tc_compute(caf, W1, b1, W2, b2, Wp3, bp, *, bb=32, interpret=False):
    b, a, d = caf.shape
    hid = W1.shape[1]
    h = W2.shape[1]
    grid = (b // bb,)

    def body(caf_ref, w1_ref, b1_ref, w2_ref, b2_ref, wp_ref, bp_ref, out_ref):
        x3 = caf_ref[...]
        x2 = x3.reshape(bb * a, d)
        h1 = jnp.dot(x2, w1_ref[...], preferred_element_type=jnp.float32)
        h1 = h1 + b1_ref[...]
        h1 = h1 * jax.nn.sigmoid(h1)
        lg = jnp.dot(h1, w2_ref[...], preferred_element_type=jnp.float32)
        lg = lg + b2_ref[...]
        m = jnp.max(lg, axis=-1, keepdims=True)
        e = jnp.exp(lg - m)
        cw = e / jnp.sum(e, axis=-1, keepdims=True)
        cw3 = cw.reshape(bb, a, h)
        acc = jnp.zeros((bb, d), jnp.float32)
        for hh in range(h):
            wsh = jnp.sum(x3 * cw3[:, :, hh][:, :, None], axis=1)
            acc = acc + jnp.dot(wsh, wp_ref[hh],
                                preferred_element_type=jnp.float32)
        out = acc + bp_ref[...]
        out_ref[...] = out * jax.nn.sigmoid(out)

    return pl.pallas_call(
        body,
        grid=grid,
        in_specs=[
            pl.BlockSpec((bb, a, d), lambda i: (i, 0, 0)),
            pl.BlockSpec((d, hid), lambda i: (0, 0)),
            pl.BlockSpec((1, hid), lambda i: (0, 0)),
            pl.BlockSpec((hid, h), lambda i: (0, 0)),
            pl.BlockSpec((1, h), lambda i: (0, 0)),
            pl.BlockSpec((h, d, d), lambda i: (0, 0, 0)),
            pl.BlockSpec((1, d), lambda i: (0, 0)),
        ],
        out_specs=pl.BlockSpec((bb, d), lambda i: (i, 0)),
        out_shape=jax.ShapeDtypeStruct((b, d), jnp.float32),
        interpret=interpret,
    )(caf, W1, b1.reshape(1, hid), W2, b2.reshape(1, h), Wp3,
      bp.reshape(1, d))


def kernel(atom_fea, crystal_atom_idx, W1, b1, W2, b2, Wp, bp):
    b, a = crystal_atom_idx.shape
    n, d = atom_fea.shape
    h = W2.shape[1]
    caf = _sc_gather(atom_fea, crystal_atom_idx.reshape(-1))
    caf = caf.reshape(b, a, d)
    wp3 = Wp.reshape(h, d, d)
    return _tc_compute(caf, W1, b1, W2, b2, wp3, bp)


# speedup vs baseline: 3.0690x; 1.0946x over previous
import functools

import jax
import jax.numpy as jnp
from jax import lax
from jax.experimental import pallas as pl
from jax.experimental.pallas import tpu as pltpu
from jax.experimental.pallas import tpu_sc as plsc


def _sc_gather(table, idx_flat):
    n_rows, d = table.shape
    r = idx_flat.shape[0]
    nc, ns = 2, 16
    nw = nc * ns
    chunk = 128
    per_w = r // nw
    j_per_w = per_w // chunk
    idx2 = idx_flat.reshape(r // chunk, chunk)

    mesh = plsc.VectorSubcoreMesh(core_axis_name="c", subcore_axis_name="s")

    @functools.partial(
        pl.kernel,
        out_type=jax.ShapeDtypeStruct((r, d), jnp.float32),
        mesh=mesh,
        scratch_types=[
            pltpu.VMEM((j_per_w, chunk), jnp.int32),
            pltpu.VMEM((2, chunk, d), jnp.float32),
            pltpu.SemaphoreType.DMA,
            pltpu.SemaphoreType.DMA,
            pltpu.SemaphoreType.DMA,
            pltpu.SemaphoreType.DMA,
        ],
    )
    def gather_kernel(table_hbm, idx_hbm, out_hbm, idx_v, rows_v,
                      g0, g1, w0, w1):
        wid = lax.axis_index("s") * nc + lax.axis_index("c")
        gsem = (g0, g1)
        wsem = (w0, w1)
        pltpu.sync_copy(idx_hbm.at[pl.ds(wid * j_per_w, j_per_w)], idx_v)

        gathers = [None, None]
        writes = [None, None]
        gathers[0] = pltpu.async_copy(
            table_hbm.at[idx_v.at[0]], rows_v.at[0], gsem[0])
        for j in range(j_per_w):
            b = j % 2
            if j + 1 < j_per_w:
                if writes[1 - b] is not None:
                    writes[1 - b].wait()
                gathers[1 - b] = pltpu.async_copy(
                    table_hbm.at[idx_v.at[j + 1]], rows_v.at[1 - b],
                    gsem[1 - b])
            gathers[b].wait()
            writes[b] = pltpu.async_copy(
                rows_v.at[b],
                out_hbm.at[pl.ds(wid * per_w + j * chunk, chunk)], wsem[b])
        for w in writes:
            if w is not None:
                w.wait()

    return gather_kernel(table, idx2)


def _tc_compute(caf, W1, b1, W2, b2, Wp3, bp, *, bb=32, interpret=False):
    b, a, d = caf.shape
    hid = W1.shape[1]
    h = W2.shape[1]
    grid = (b // bb,)

    def body(caf_ref, w1_ref, b1_ref, w2_ref, b2_ref, wp_ref, bp_ref, out_ref):
        x3 = caf_ref[...]
        x2 = x3.reshape(bb * a, d)
        h1 = jnp.dot(x2, w1_ref[...], preferred_element_type=jnp.float32)
        h1 = h1 + b1_ref[...]
        h1 = h1 * jax.nn.sigmoid(h1)
        lg = jnp.dot(h1, w2_ref[...], preferred_element_type=jnp.float32)
        lg = lg + b2_ref[...]
        m = jnp.max(lg, axis=-1, keepdims=True)
        e = jnp.exp(lg - m)
        cw = e / jnp.sum(e, axis=-1, keepdims=True)
        cw3 = cw.reshape(bb, a, h)
        acc = jnp.zeros((bb, d), jnp.float32)
        for hh in range(h):
            wsh = jnp.sum(x3 * cw3[:, :, hh][:, :, None], axis=1)
            acc = acc + jnp.dot(wsh, wp_ref[hh],
                                preferred_element_type=jnp.float32)
        out = acc + bp_ref[...]
        out_ref[...] = out * jax.nn.sigmoid(out)

    return pl.pallas_call(
        body,
        grid=grid,
        in_specs=[
            pl.BlockSpec((bb, a, d), lambda i: (i, 0, 0)),
            pl.BlockSpec((d, hid), lambda i: (0, 0)),
            pl.BlockSpec((1, hid), lambda i: (0, 0)),
            pl.BlockSpec((hid, h), lambda i: (0, 0)),
            pl.BlockSpec((1, h), lambda i: (0, 0)),
            pl.BlockSpec((h, d, d), lambda i: (0, 0, 0)),
            pl.BlockSpec((1, d), lambda i: (0, 0)),
        ],
        out_specs=pl.BlockSpec((bb, d), lambda i: (i, 0)),
        out_shape=jax.ShapeDtypeStruct((b, d), jnp.float32),
        interpret=interpret,
    )(caf, W1, b1.reshape(1, hid), W2, b2.reshape(1, h), Wp3,
      bp.reshape(1, d))


def kernel(atom_fea, crystal_atom_idx, W1, b1, W2, b2, Wp, bp):
    b, a = crystal_atom_idx.shape
    n, d = atom_fea.shape
    h = W2.shape[1]
    caf = _sc_gather(atom_fea, crystal_atom_idx.reshape(-1))
    caf = caf.reshape(b, a, d)
    wp3 = Wp.reshape(h, d, d)
    return _tc_compute(caf, W1, b1, W2, b2, wp3, bp)


# speedup vs baseline: 4.3973x; 1.4328x over previous
import functools

import jax
import jax.numpy as jnp
from jax import lax
from jax.experimental import pallas as pl
from jax.experimental.pallas import tpu as pltpu
from jax.experimental.pallas import tpu_sc as plsc


def _sc_gather(table, idx_flat):
    n_rows, d = table.shape
    r = idx_flat.shape[0]
    nc, ns = 2, 16
    nw = nc * ns
    chunk = 128
    per_w = r // nw
    j_per_w = per_w // chunk
    idx2 = idx_flat.reshape(r // chunk, chunk)

    mesh = plsc.VectorSubcoreMesh(core_axis_name="c", subcore_axis_name="s")

    @functools.partial(
        pl.kernel,
        out_type=jax.ShapeDtypeStruct((r, d), jnp.float32),
        mesh=mesh,
        scratch_types=[
            pltpu.VMEM((j_per_w, chunk), jnp.int32),
            pltpu.VMEM((2, chunk, d), jnp.float32),
            pltpu.SemaphoreType.DMA,
            pltpu.SemaphoreType.DMA,
            pltpu.SemaphoreType.DMA,
            pltpu.SemaphoreType.DMA,
        ],
    )
    def gather_kernel(table_hbm, idx_hbm, out_hbm, idx_v, rows_v,
                      g0, g1, w0, w1):
        wid = lax.axis_index("s") * nc + lax.axis_index("c")
        gsem = (g0, g1)
        wsem = (w0, w1)
        pltpu.sync_copy(idx_hbm.at[pl.ds(wid * j_per_w, j_per_w)], idx_v)

        gathers = [None, None]
        writes = [None, None]
        gathers[0] = pltpu.async_copy(
            table_hbm.at[idx_v.at[0]], rows_v.at[0], gsem[0])
        for j in range(j_per_w):
            b = j % 2
            if j + 1 < j_per_w:
                if writes[1 - b] is not None:
                    writes[1 - b].wait()
                gathers[1 - b] = pltpu.async_copy(
                    table_hbm.at[idx_v.at[j + 1]], rows_v.at[1 - b],
                    gsem[1 - b])
            gathers[b].wait()
            writes[b] = pltpu.async_copy(
                rows_v.at[b],
                out_hbm.at[pl.ds(wid * per_w + j * chunk, chunk)], wsem[b])
        for w in writes:
            if w is not None:
                w.wait()

    return gather_kernel(table, idx2)


def _tc_compute(caf, W1, b1, W2, b2, Wp3, bp, *, bb=32, interpret=False):
    b, a, d = caf.shape
    hid = W1.shape[1]
    h = W2.shape[1]
    grid = (b // bb,)

    def body(caf_ref, w1_ref, b1_ref, w2_ref, b2_ref, wp_ref, bp_ref, out_ref):
        x2 = caf_ref[...].reshape(bb * a, d)
        h1 = jnp.dot(x2, w1_ref[...], preferred_element_type=jnp.float32)
        h1 = h1 + b1_ref[...]
        h1 = h1 * jax.nn.sigmoid(h1)
        lgt = lax.dot_general(w2_ref[...], h1, (((0,), (1,)), ((), ())),
                              preferred_element_type=jnp.float32)
        lgt = lgt + b2_ref[...]
        m = jnp.max(lgt, axis=0, keepdims=True)
        e = jnp.exp(lgt - m)
        cwt = e / jnp.sum(e, axis=0, keepdims=True)
        col = lax.broadcasted_iota(jnp.int32, (bb, bb * a), 1)
        row = lax.broadcasted_iota(jnp.int32, (bb, bb * a), 0)
        mask = (col // a) == row
        acc = jnp.zeros((bb, d), jnp.float32)
        for hh in range(h):
            cw = jnp.where(mask, cwt[hh][None, :], 0.0)
            wsh = jnp.dot(cw, x2, preferred_element_type=jnp.float32)
            acc = acc + jnp.dot(wsh, wp_ref[hh],
                                preferred_element_type=jnp.float32)
        out = acc + bp_ref[...]
        out_ref[...] = out * jax.nn.sigmoid(out)

    return pl.pallas_call(
        body,
        grid=grid,
        in_specs=[
            pl.BlockSpec((bb, a, d), lambda i: (i, 0, 0)),
            pl.BlockSpec((d, hid), lambda i: (0, 0)),
            pl.BlockSpec((1, hid), lambda i: (0, 0)),
            pl.BlockSpec((hid, h), lambda i: (0, 0)),
            pl.BlockSpec((h, 1), lambda i: (0, 0)),
            pl.BlockSpec((h, d, d), lambda i: (0, 0, 0)),
            pl.BlockSpec((1, d), lambda i: (0, 0)),
        ],
        out_specs=pl.BlockSpec((bb, d), lambda i: (i, 0)),
        out_shape=jax.ShapeDtypeStruct((b, d), jnp.float32),
        interpret=interpret,
    )(caf, W1, b1.reshape(1, hid), W2, b2.reshape(h, 1), Wp3,
      bp.reshape(1, d))


def kernel(atom_fea, crystal_atom_idx, W1, b1, W2, b2, Wp, bp):
    b, a = crystal_atom_idx.shape
    n, d = atom_fea.shape
    h = W2.shape[1]
    caf = _sc_gather(atom_fea, crystal_atom_idx.reshape(-1))
    caf = caf.reshape(b, a, d)
    wp3 = Wp.reshape(h, d, d)
    return _tc_compute(caf, W1, b1, W2, b2, wp3, bp)
